# Initial kernel scaffold; baseline (speedup 1.0000x reference)
import jax
import jax.numpy as jnp
from jax.experimental import pallas as pl


def kernel(x, edge_index, edge_attr, W, We, att, b):
    raise NotImplementedError("write your pallas kernel here")


# speedup vs baseline: 7.4817x; 7.4817x over previous
import functools

import jax
import jax.numpy as jnp
from jax import lax
from jax.experimental import pallas as pl
from jax.experimental.pallas import tpu as pltpu
from jax.experimental.pallas import tpu_sc as plsc

N = 10000
E = 320000
D = 128
C = 128
DE = 16
EE = 4
NEG_SLOPE = 0.2

NW = 32
K = 128
NCHUNK = 80
EPW = K * NCHUNK
E_PAD = NW * EPW
N_PAD = 10240
RPS = N_PAD // 16


def _a1_body(x_ref, w_ref, ai_ref, aj_ref, xp_ref, si_ref, sj_ref,
             mi_ref, mj_ref):
    xp = jnp.dot(x_ref[...], w_ref[...], preferred_element_type=jnp.float32)
    xp_ref[...] = xp
    si = jnp.sum(xp * ai_ref[...], axis=1, keepdims=True)
    sj = jnp.sum(xp * aj_ref[...], axis=1, keepdims=True)
    si_ref[...] = si
    sj_ref[...] = sj
    mi_ref[...] = jnp.broadcast_to(jnp.max(si), (1, 1, 8))
    mj_ref[...] = jnp.broadcast_to(jnp.max(sj), (1, 1, 8))


def _a2_body(ea_ref, we_ref, ae_ref, esc_ref, me_ref):
    wv = jnp.sum(we_ref[...] * ae_ref[...], axis=1)
    esc = jnp.sum(ea_ref[...] * wv[None, :], axis=1, keepdims=True)
    esc_ref[...] = esc
    me_ref[...] = jnp.broadcast_to(jnp.max(esc), (1, 1, 8))


def _c_body(num_ref, den_ref, b_ref, out_ref):
    s = num_ref[0] + num_ref[1]
    d = den_ref[0] + den_ref[1]
    out_ref[...] = s / (d + 1e-16) + b_ref[...]


def _sc_body(src_hbm, dst_hbm, esc_hbm, si_hbm, sj_hbm, b16_hbm, xp_hbm,
             num_out, den_out,
             si_v, sj_v, b16_v, src_f, dst_f, esc_f, w_c,
             rows2, num_s, den_s, sem):
    c = lax.axis_index("c")
    s = lax.axis_index("s")
    wid = s * 2 + c

    pltpu.sync_copy(si_hbm, si_v)
    pltpu.sync_copy(sj_hbm, sj_v)
    pltpu.sync_copy(b16_hbm, b16_v)

    zv = jnp.zeros((16,), jnp.float32)

    def zbody(r, _):
        w_c[pl.ds(r * 16, 16)] = zv

        def zrow(jj, _2):
            rows2[2 * r, pl.ds(jj * 16, 16)] = zv
            rows2[2 * r + 1, pl.ds(jj * 16, 16)] = zv
            return 0

        lax.fori_loop(0, 8, zrow, 0)
        return 0

    lax.fori_loop(0, K // 2, zbody, 0)
    for t in range(RPS // K):
        pltpu.sync_copy(rows2, num_s.at[pl.ds(s * RPS + t * K, K)])
        pltpu.sync_copy(w_c, den_s.at[pl.ds(s * RPS + t * K, K)])
    plsc.subcore_barrier()

    bv = b16_v[...]
    ebase = wid * EPW

    def chunk(i, carry):
        off = ebase + i * K
        pltpu.sync_copy(src_hbm.at[pl.ds(off, K)], src_f)
        cp = pltpu.async_copy(xp_hbm.at[src_f], rows2, sem)
        pltpu.sync_copy(dst_hbm.at[pl.ds(off, K)], dst_f)
        pltpu.sync_copy(esc_hbm.at[pl.ds(off, K)], esc_f)

        def wbody(j, _):
            sl = pl.ds(j * 16, 16)
            gi = plsc.load_gather(si_v, [dst_f[sl]])
            gj = plsc.load_gather(sj_v, [src_f[sl]])
            a = gi + gj + esc_f[sl]
            a = jnp.where(a >= 0.0, a, a * NEG_SLOPE)
            w_c[sl] = jnp.exp(a - bv)
            return 0

        lax.fori_loop(0, K // 16, wbody, 0)
        cp.wait()

        def sbody(r, _):
            idx = jnp.full((16,), r, jnp.int32)
            wv = plsc.load_gather(w_c, [idx])
            for jj in range(8):
                sl = pl.ds(jj * 16, 16)
                rows2[r, sl] = rows2[r, sl] * wv
            return 0

        lax.fori_loop(0, K, sbody, 0)
        pltpu.sync_copy(rows2, num_s.at[dst_f], add=True)
        pltpu.sync_copy(w_c, den_s.at[dst_f], add=True)
        return 0

    lax.fori_loop(0, NCHUNK, chunk, 0)
    plsc.subcore_barrier()

    pltpu.sync_copy(num_s.at[pl.ds(s * RPS, RPS)],
                    num_out.at[c, pl.ds(s * RPS, RPS)])

    @pl.when(s == 0)
    def _():
        pltpu.sync_copy(den_s, den_out.at[c, 0])


_sc_call = functools.partial(
    pl.kernel,
    out_type=[
        jax.ShapeDtypeStruct((2, N_PAD, C), jnp.float32),
        jax.ShapeDtypeStruct((2, 1, N_PAD), jnp.float32),
    ],
    mesh=plsc.VectorSubcoreMesh(core_axis_name="c", subcore_axis_name="s"),
    compiler_params=pltpu.CompilerParams(needs_layout_passes=False),
    scratch_types=[
        pltpu.VMEM((N,), jnp.float32),
        pltpu.VMEM((N,), jnp.float32),
        pltpu.VMEM((16,), jnp.float32),
        pltpu.VMEM((K,), jnp.int32),
        pltpu.VMEM((K,), jnp.int32),
        pltpu.VMEM((K,), jnp.float32),
        pltpu.VMEM((K,), jnp.float32),
        pltpu.VMEM((K, C), jnp.float32),
        pltpu.VMEM_SHARED((N_PAD, C), jnp.float32),
        pltpu.VMEM_SHARED((N_PAD,), jnp.float32),
        pltpu.SemaphoreType.DMA,
    ],
)(_sc_body)


def kernel(x, edge_index, edge_attr, W, We, att, b):
    ai = att[0, :, :C].astype(jnp.float32)
    aj = att[0, :, C:2 * C].astype(jnp.float32)
    ae = att[0, :, 2 * C:].astype(jnp.float32)

    xp, si, sj, mi, mj = pl.pallas_call(
        _a1_body,
        grid=(25,),
        in_specs=[
            pl.BlockSpec((400, D), lambda i: (i, 0)),
            pl.BlockSpec((D, C), lambda i: (0, 0)),
            pl.BlockSpec((1, C), lambda i: (0, 0)),
            pl.BlockSpec((1, C), lambda i: (0, 0)),
        ],
        out_specs=[
            pl.BlockSpec((400, C), lambda i: (i, 0)),
            pl.BlockSpec((400, 1), lambda i: (i, 0)),
            pl.BlockSpec((400, 1), lambda i: (i, 0)),
            pl.BlockSpec((1, 1, 8), lambda i: (i, 0, 0)),
            pl.BlockSpec((1, 1, 8), lambda i: (i, 0, 0)),
        ],
        out_shape=[
            jax.ShapeDtypeStruct((N, C), jnp.float32),
            jax.ShapeDtypeStruct((N, 1), jnp.float32),
            jax.ShapeDtypeStruct((N, 1), jnp.float32),
            jax.ShapeDtypeStruct((25, 1, 8), jnp.float32),
            jax.ShapeDtypeStruct((25, 1, 8), jnp.float32),
        ],
    )(x, W, ai, aj)

    esc, me = pl.pallas_call(
        _a2_body,
        grid=(50,),
        in_specs=[
            pl.BlockSpec((6400, DE), lambda i: (i, 0)),
            pl.BlockSpec((DE, EE), lambda i: (0, 0)),
            pl.BlockSpec((1, EE), lambda i: (0, 0)),
        ],
        out_specs=[
            pl.BlockSpec((6400, 1), lambda i: (i, 0)),
            pl.BlockSpec((1, 1, 8), lambda i: (i, 0, 0)),
        ],
        out_shape=[
            jax.ShapeDtypeStruct((E, 1), jnp.float32),
            jax.ShapeDtypeStruct((50, 1, 8), jnp.float32),
        ],
    )(edge_attr, We, ae)

    braw = jnp.max(mi) + jnp.max(mj) + jnp.max(me)
    bnd = jnp.where(braw >= 0.0, braw, braw * NEG_SLOPE)
    b16 = jnp.full((16,), bnd, jnp.float32)

    pad = E_PAD - E
    ipad = jnp.zeros((pad,), jnp.int32)
    src = jnp.concatenate([edge_index[0].astype(jnp.int32), ipad])
    dst = jnp.concatenate([edge_index[1].astype(jnp.int32), ipad])
    escp = jnp.concatenate([esc.reshape(E),
                            jnp.full((pad,), -1e30, jnp.float32)])

    num2, den2 = _sc_call(
        src, dst, escp, si.reshape(N), sj.reshape(N), b16, xp,
    )

    out = pl.pallas_call(
        _c_body,
        grid=(25,),
        in_specs=[
            pl.BlockSpec((2, 400, C), lambda i: (0, i, 0)),
            pl.BlockSpec((2, 400, 1), lambda i: (0, i, 0)),
            pl.BlockSpec((1, C), lambda i: (0, 0)),
        ],
        out_specs=pl.BlockSpec((400, C), lambda i: (i, 0)),
        out_shape=jax.ShapeDtypeStruct((N, C), jnp.float32),
    )(num2[:, :N], den2.reshape(2, N_PAD, 1)[:, :N], b.reshape(1, C))
    return out
